# Initial kernel scaffold; baseline (speedup 1.0000x reference)
import jax
import jax.numpy as jnp
from jax.experimental import pallas as pl


def kernel(x, edge_index, edge_attr, pri_idx, pri_seg, cond_idx, cond_seg, W_e, W1, b1):
    raise NotImplementedError("write your pallas kernel here")


# speedup vs baseline: 1.6697x; 1.6697x over previous
import functools

import jax
import jax.numpy as jnp
from jax import lax
from jax.experimental import pallas as pl
from jax.experimental.pallas import tpu as pltpu
import jax.experimental.pallas.tpu_sc as plsc

N = 10000
E = 160000
D = 256
DE = 16
B = 16
NP = 6144
NC = 2048

NCORE = 2
NSUB = 16
DH = D // NCORE
EPT = E // NSUB
CH = 80
NCHUNK = EPT // CH
NPAD = 10240
RPS = NPAD // NSUB


def _s1_body(src_hbm, dst_hbm, xr_hbm, aggx_out, idx_s, idx_d, rows,
             accA, sem):
    c = lax.axis_index("c")
    s = lax.axis_index("s")
    zero16 = jnp.zeros((16,), jnp.float32)

    def zrow(i, _):
        for j in range(DH // 16):
            rows[i, pl.ds(j * 16, 16)] = zero16
        return _
    lax.fori_loop(0, CH, zrow, None)
    for k in range(RPS // CH):
        pltpu.sync_copy(rows, accA.at[pl.ds(s * RPS + k * CH, CH)])
    plsc.subcore_barrier()

    base0 = s * EPT
    coff = c * N

    def chunk(k, _):
        base = base0 + k * CH
        pltpu.sync_copy(src_hbm.at[pl.ds(base, CH)], idx_s)
        pltpu.sync_copy(dst_hbm.at[pl.ds(base, CH)], idx_d)
        coffv = jnp.full((16,), coff, jnp.int32)
        for j in range(CH // 16):
            idx_s[pl.ds(j * 16, 16)] = idx_s[pl.ds(j * 16, 16)] + coffv
        pltpu.async_copy(xr_hbm.at[idx_s], rows, sem).wait()
        pltpu.sync_copy(rows, accA.at[idx_d], add=True)
        return _

    lax.fori_loop(0, NCHUNK, chunk, None)
    plsc.subcore_barrier()

    @pl.when(s < NSUB - 1)
    def _():
        pltpu.sync_copy(accA.at[pl.ds(s * RPS, RPS)],
                        aggx_out.at[pl.ds(c * N + s * RPS, RPS)])

    @pl.when(s == NSUB - 1)
    def _():
        pltpu.sync_copy(accA.at[pl.ds((NSUB - 1) * RPS, N - (NSUB - 1) * RPS)],
                        aggx_out.at[pl.ds(c * N + (NSUB - 1) * RPS,
                                          N - (NSUB - 1) * RPS)])


@functools.cache
def _get_s1():
  return functools.partial(
    pl.kernel,
    out_type=jax.ShapeDtypeStruct((2 * N, DH), jnp.float32),
    mesh=plsc.VectorSubcoreMesh(core_axis_name="c", subcore_axis_name="s",
                                num_cores=NCORE, num_subcores=NSUB),
    scratch_types=[
        pltpu.VMEM((CH,), jnp.int32),
        pltpu.VMEM((CH,), jnp.int32),
        pltpu.VMEM((CH, DH), jnp.float32),
        pltpu.VMEM_SHARED((NPAD, DH), jnp.float32),
        pltpu.SemaphoreType.DMA,
    ],
  )(_s1_body)


EPT2 = E // (NCORE * NSUB)
NFULL = EPT2 // CH
TAIL = EPT2 - NFULL * CH


def _s1b_body(dst_hbm, ea_hbm, eagg_out, idx_d, idx_t, ea_buf, rows, accE,
              sem):
    c = lax.axis_index("c")
    s = lax.axis_index("s")
    zero16 = jnp.zeros((16,), jnp.float32)

    def zrow(i, _):
        for j in range(DH // 16):
            rows[i, pl.ds(j * 16, 16)] = zero16
        return _
    lax.fori_loop(0, CH, zrow, None)
    for k in range(RPS // CH):
        pltpu.sync_copy(rows, accE.at[pl.ds(s * RPS + k * CH, CH)])
    plsc.subcore_barrier()

    ebase = (c * NSUB + s) * EPT2

    def chunk(k, _):
        base = ebase + k * CH
        pltpu.sync_copy(dst_hbm.at[pl.ds(base, CH)], idx_d)
        pltpu.sync_copy(ea_hbm.at[pl.ds(base, CH)], ea_buf)

        def crow(j, __):
            rows[j, pl.ds(0, DE)] = ea_buf[j, :]
            return __
        lax.fori_loop(0, CH, crow, None)
        pltpu.sync_copy(rows, accE.at[idx_d], add=True)
        return _

    lax.fori_loop(0, NFULL, chunk, None)

    tbase = ebase + NFULL * CH
    pltpu.sync_copy(dst_hbm.at[pl.ds(tbase, TAIL)], idx_t)
    pltpu.sync_copy(ea_hbm.at[pl.ds(tbase, TAIL)], ea_buf.at[pl.ds(0, TAIL)])

    def crow_t(j, _):
        rows[j, pl.ds(0, DE)] = ea_buf[j, :]
        return _
    lax.fori_loop(0, TAIL, crow_t, None)
    pltpu.sync_copy(rows.at[pl.ds(0, TAIL)], accE.at[idx_t], add=True)
    plsc.subcore_barrier()

    @pl.when(s < NSUB - 1)
    def _():
        pltpu.sync_copy(accE.at[pl.ds(s * RPS, RPS)],
                        eagg_out.at[pl.ds(c * N + s * RPS, RPS)])

    @pl.when(s == NSUB - 1)
    def _():
        pltpu.sync_copy(accE.at[pl.ds((NSUB - 1) * RPS, N - (NSUB - 1) * RPS)],
                        eagg_out.at[pl.ds(c * N + (NSUB - 1) * RPS,
                                          N - (NSUB - 1) * RPS)])


@functools.cache
def _get_s1b():
  return functools.partial(
    pl.kernel,
    out_type=jax.ShapeDtypeStruct((2 * N, DH), jnp.float32),
    mesh=plsc.VectorSubcoreMesh(core_axis_name="c", subcore_axis_name="s",
                                num_cores=NCORE, num_subcores=NSUB),
    scratch_types=[
        pltpu.VMEM((CH,), jnp.int32),
        pltpu.VMEM((TAIL,), jnp.int32),
        pltpu.VMEM((CH, DE), jnp.float32),
        pltpu.VMEM((CH, DH), jnp.float32),
        pltpu.VMEM_SHARED((NPAD, DH), jnp.float32),
        pltpu.SemaphoreType.DMA,
    ],
  )(_s1b_body)


def _s2_body(x_ref, a0_ref, a1_ref, e0_ref, e1_ref, we_ref, w1_ref, b1_ref,
             out_ref):
    agg = jnp.concatenate([a0_ref[0], a1_ref[0]], axis=-1)
    a = x_ref[...] + agg
    eagg = e0_ref[0] + e1_ref[0]
    we1 = jnp.dot(we_ref[...], w1_ref[...], preferred_element_type=jnp.float32)
    acc = jnp.dot(a, w1_ref[...], preferred_element_type=jnp.float32)
    acc = acc + jnp.dot(eagg, we1, preferred_element_type=jnp.float32)
    out_ref[...] = jnp.maximum(acc + b1_ref[...], 0.0)


def _s2(x, aggx2, eagg2, W_e, W1, b1):
    blk = 200
    grid = N // blk
    return pl.pallas_call(
        _s2_body,
        grid=(grid,),
        in_specs=[
            pl.BlockSpec((blk, D), lambda i: (i, 0)),
            pl.BlockSpec((1, blk, DH), lambda i: (0, i, 0)),
            pl.BlockSpec((1, blk, DH), lambda i: (1, i, 0)),
            pl.BlockSpec((1, blk, DE), lambda i: (0, i, 0)),
            pl.BlockSpec((1, blk, DE), lambda i: (1, i, 0)),
            pl.BlockSpec((DE, D), lambda i: (0, 0)),
            pl.BlockSpec((D, D), lambda i: (0, 0)),
            pl.BlockSpec((1, D), lambda i: (0, 0)),
        ],
        out_specs=pl.BlockSpec((blk, D), lambda i: (i, 0)),
        out_shape=jax.ShapeDtypeStruct((N, D), jnp.float32),
    )(x, aggx2, aggx2, eagg2, eagg2, W_e, W1, b1.reshape(1, D))


NOUT = NP + NC
RPT = NOUT // (NCORE * NSUB)
OCH = 64
CPT = NC // NSUB


def _s3_body(nrep_hbm, nidx_hbm, w_hbm, rseg_hbm, cidx_hbm, cseg_hbm,
             cw_hbm, out_hbm,
             cidxb, csegb, cwb, crows, tacc, iota16, cacc,
             nbuf, wbuf, rsegb, g, ob, accP, sem):
    c = lax.axis_index("c")
    s = lax.axis_index("s")
    zero16 = jnp.zeros((16,), jnp.float32)
    iota16[...] = lax.iota(jnp.int32, 16)

    def ztacc(i, _):
        for v in range(D // 16):
            tacc[i, pl.ds(v * 16, 16)] = zero16
        return _
    lax.fori_loop(0, B, ztacc, None)

    cb = s * CPT
    pltpu.sync_copy(cidx_hbm.at[pl.ds(cb, CPT)], cidxb)
    pltpu.sync_copy(cseg_hbm.at[pl.ds(cb, CPT)], csegb)
    pltpu.sync_copy(cw_hbm.at[pl.ds(cb, CPT)], cwb)
    pltpu.async_copy(nrep_hbm.at[cidxb], crows, sem).wait()

    def pool_row(j, _):
        segv = plsc.load_gather(csegb, [jnp.full((16,), j, jnp.int32)])
        wv = plsc.load_gather(cwb, [jnp.full((16,), j, jnp.int32)])
        for v in range(D // 16):
            val = crows[j, pl.ds(v * 16, 16)] * wv
            plsc.addupdate_scatter(tacc, [segv, iota16[...] + v * 16], val)
        return _
    lax.fori_loop(0, CPT, pool_row, None)

    pltpu.sync_copy(tacc, accP.at[s])

    def zcacc(i, _):
        for v in range(D // 16):
            cacc[i, pl.ds(v * 16, 16)] = zero16
        return _
    lax.fori_loop(0, 32, zcacc, None)
    plsc.subcore_barrier()
    for t in range(NSUB):
        pltpu.sync_copy(accP.at[t], tacc)

        def addrow(i, _):
            for v in range(D // 16):
                cacc[i, pl.ds(v * 16, 16)] = (cacc[i, pl.ds(v * 16, 16)]
                                              + tacc[i, pl.ds(v * 16, 16)])
            return _
        lax.fori_loop(0, B, addrow, None)

    rb = (c * NSUB + s) * RPT

    for ch in range(RPT // OCH):
        ob_base = rb + ch * OCH
        pltpu.sync_copy(nidx_hbm.at[pl.ds(ob_base, OCH)], nbuf)
        pltpu.sync_copy(w_hbm.at[pl.ds(ob_base, OCH)], wbuf)
        pltpu.sync_copy(rseg_hbm.at[pl.ds(ob_base, OCH)], rsegb)
        pltpu.async_copy(nrep_hbm.at[nbuf], g, sem).wait()

        def row(j, _):
            wv = plsc.load_gather(wbuf, [jnp.full((16,), j, jnp.int32)])
            rsv = plsc.load_gather(rsegb, [jnp.full((16,), j, jnp.int32)])
            for v in range(D // 16):
                ob[j, pl.ds(v * 16, 16)] = g[j, pl.ds(v * 16, 16)] * wv
            for v in range(D // 16):
                rv = plsc.load_gather(cacc, [rsv, iota16[...] + v * 16])
                ob[j, pl.ds(D + v * 16, 16)] = rv
            return _
        lax.fori_loop(0, OCH, row, None)
        pltpu.sync_copy(ob, out_hbm.at[pl.ds(ob_base, OCH)])


@functools.cache
def _get_s3():
  return functools.partial(
    pl.kernel,
    out_type=jax.ShapeDtypeStruct((NOUT, 2 * D), jnp.float32),
    mesh=plsc.VectorSubcoreMesh(core_axis_name="c", subcore_axis_name="s",
                                num_cores=NCORE, num_subcores=NSUB),
    compiler_params=pltpu.CompilerParams(needs_layout_passes=False),
    scratch_types=[
        pltpu.VMEM((CPT,), jnp.int32),
        pltpu.VMEM((CPT,), jnp.int32),
        pltpu.VMEM((CPT,), jnp.float32),
        pltpu.VMEM((CPT, D), jnp.float32),
        pltpu.VMEM((B, D), jnp.float32),
        pltpu.VMEM((16,), jnp.int32),
        pltpu.VMEM((32, D), jnp.float32),
        pltpu.VMEM((OCH,), jnp.int32),
        pltpu.VMEM((OCH,), jnp.float32),
        pltpu.VMEM((OCH,), jnp.int32),
        pltpu.VMEM((OCH, D), jnp.float32),
        pltpu.VMEM((OCH, 2 * D), jnp.float32),
        pltpu.VMEM_SHARED((NSUB, B, D), jnp.float32),
        pltpu.SemaphoreType.DMA,
    ],
  )(_s3_body)


def kernel(x, edge_index, edge_attr, pri_idx, pri_seg, cond_idx, cond_seg,
           W_e, W1, b1):
    src = edge_index[0].astype(jnp.int32)
    dst = edge_index[1].astype(jnp.int32)
    xr = x.reshape(N, 2, DH).transpose(1, 0, 2).reshape(2 * N, DH)

    aggx = _get_s1()(src, dst, xr)
    eaggw = _get_s1b()(dst, edge_attr)
    eagg2 = eaggw.reshape(2, N, DH)[:, :, :DE]
    node_rep = _s2(x, aggx.reshape(2, N, DH), eagg2, W_e, W1, b1)

    pri_seg = pri_seg.astype(jnp.int32)
    cond_seg = cond_seg.astype(jnp.int32)
    bp = jnp.searchsorted(pri_seg, jnp.arange(B + 1, dtype=jnp.int32))
    bc = jnp.searchsorted(cond_seg, jnp.arange(B + 1, dtype=jnp.int32))
    cnt_p = (bp[1:] - bp[:-1]).astype(jnp.int32)
    cnt_c = (bc[1:] - bc[:-1]).astype(jnp.int32)
    start_p = bp[:-1].astype(jnp.int32)
    tot = cnt_p + cnt_c
    offsets = jnp.concatenate([jnp.zeros((1,), jnp.int32),
                               jnp.cumsum(tot)[:-1].astype(jnp.int32)])
    r = jnp.arange(NOUT, dtype=jnp.int32)
    seg_r = jnp.searchsorted(jnp.cumsum(tot).astype(jnp.int32), r,
                             side="right").astype(jnp.int32)
    local = r - offsets[seg_r]
    is_pri = local < cnt_p[seg_r]
    psrc = jnp.clip(start_p[seg_r] + local, 0, NP - 1)
    nidx = jnp.where(is_pri, pri_idx[psrc].astype(jnp.int32), 0)
    w = is_pri.astype(jnp.float32)
    rseg = jnp.where(is_pri, seg_r, 16).astype(jnp.int32)
    cw = (1.0 / jnp.maximum(cnt_c, 1).astype(jnp.float32))[cond_seg]

    out = _get_s3()(node_rep, nidx, w, rseg, cond_idx.astype(jnp.int32),
                    cond_seg, cw)
    return out


# speedup vs baseline: 2.0160x; 1.2074x over previous
import functools

import jax
import jax.numpy as jnp
from jax import lax
from jax.experimental import pallas as pl
from jax.experimental.pallas import tpu as pltpu
import jax.experimental.pallas.tpu_sc as plsc

N = 10000
E = 160000
D = 256
DE = 16
B = 16
NP = 6144
NC = 2048

NCORE = 2
NSUB = 16
DH = D // NCORE
EPT = E // NSUB
CH = 80
NCHUNK = EPT // CH
NPAD = 10240
RPS = NPAD // NSUB


def _s1_body(sd_hbm, xr_hbm, aggx_out, ir0, ir1, is0, is1, id0, id1,
             r0, r1, accA, semi, semg):
    idxraw = (ir0, ir1)
    idx_s = (is0, is1)
    idx_d = (id0, id1)
    rows = (r0, r1)
    c = lax.axis_index("c")
    s = lax.axis_index("s")
    zero16 = jnp.zeros((16,), jnp.float32)

    def zrow(i, _):
        for j in range(DH // 16):
            rows[0][i, pl.ds(j * 16, 16)] = zero16
        return _
    lax.fori_loop(0, CH, zrow, None)
    for k in range(RPS // CH):
        pltpu.sync_copy(rows[0], accA.at[pl.ds(s * RPS + k * CH, CH)])
    plsc.subcore_barrier()

    row0 = s * NCHUNK
    coff = c * N

    def load_idx(k, p):
        pltpu.async_copy(sd_hbm.at[pl.ds((row0 + k) * 2 * CH, 2 * CH)],
                         idxraw[p], semi)

    def build_idx(p):
        coffv = jnp.full((16,), coff, jnp.int32)
        for j in range(CH // 16):
            idx_s[p][pl.ds(j * 16, 16)] = (idxraw[p][pl.ds(j * 16, 16)]
                                           + coffv)
            idx_d[p][pl.ds(j * 16, 16)] = idxraw[p][pl.ds(CH + j * 16, 16)]

    load_idx(0, 0)
    pltpu.make_async_copy(sd_hbm.at[pl.ds(row0 * 2 * CH, 2 * CH)],
                          idxraw[0], semi).wait()
    build_idx(0)
    load_idx(1, 1)
    pltpu.async_copy(xr_hbm.at[idx_s[0]], rows[0], semg)

    def pair(kp, _):
        for par in (1, 0):
            k = 2 * kp + (1 if par == 1 else 2)
            q = 1 - par
            pltpu.make_async_copy(sd_hbm.at[pl.ds((row0 + k) * 2 * CH,
                                                  2 * CH)],
                                  idxraw[par], semi).wait()
            build_idx(par)
            load_idx(k + 1, q)
            pltpu.make_async_copy(xr_hbm.at[idx_s[q]], rows[q], semg).wait()
            pltpu.async_copy(xr_hbm.at[idx_s[par]], rows[par], semg)
            pltpu.sync_copy(rows[q], accA.at[idx_d[q]], add=True)
        return _

    lax.fori_loop(0, (NCHUNK - 1) // 2, pair, None)
    pltpu.make_async_copy(sd_hbm.at[pl.ds(row0 * 2 * CH, 2 * CH)],
                          idxraw[1], semi).wait()
    pltpu.make_async_copy(xr_hbm.at[idx_s[0]], rows[0], semg).wait()
    pltpu.sync_copy(rows[0], accA.at[idx_d[0]], add=True)
    plsc.subcore_barrier()

    @pl.when(s < NSUB - 1)
    def _():
        pltpu.sync_copy(accA.at[pl.ds(s * RPS, RPS)],
                        aggx_out.at[pl.ds(c * N + s * RPS, RPS)])

    @pl.when(s == NSUB - 1)
    def _():
        pltpu.sync_copy(accA.at[pl.ds((NSUB - 1) * RPS, N - (NSUB - 1) * RPS)],
                        aggx_out.at[pl.ds(c * N + (NSUB - 1) * RPS,
                                          N - (NSUB - 1) * RPS)])


@functools.cache
def _get_s1():
  return functools.partial(
    pl.kernel,
    out_type=jax.ShapeDtypeStruct((2 * N, DH), jnp.float32),
    mesh=plsc.VectorSubcoreMesh(core_axis_name="c", subcore_axis_name="s",
                                num_cores=NCORE, num_subcores=NSUB),
    scratch_types=[
        pltpu.VMEM((2 * CH,), jnp.int32),
        pltpu.VMEM((2 * CH,), jnp.int32),
        pltpu.VMEM((CH,), jnp.int32),
        pltpu.VMEM((CH,), jnp.int32),
        pltpu.VMEM((CH,), jnp.int32),
        pltpu.VMEM((CH,), jnp.int32),
        pltpu.VMEM((CH, DH), jnp.float32),
        pltpu.VMEM((CH, DH), jnp.float32),
        pltpu.VMEM_SHARED((NPAD, DH), jnp.float32),
        pltpu.SemaphoreType.DMA,
        pltpu.SemaphoreType.DMA,
    ],
  )(_s1_body)


EPT2 = E // (NCORE * NSUB)
NFULL = EPT2 // CH
TAIL = EPT2 - NFULL * CH


def _s1b_body(dst_hbm, ea_hbm, eagg_out, idx_d, idx_t, ea_buf, rows, accE,
              sem):
    c = lax.axis_index("c")
    s = lax.axis_index("s")
    zero16 = jnp.zeros((16,), jnp.float32)

    def zrow(i, _):
        for j in range(DH // 16):
            rows[i, pl.ds(j * 16, 16)] = zero16
        return _
    lax.fori_loop(0, CH, zrow, None)
    for k in range(RPS // CH):
        pltpu.sync_copy(rows, accE.at[pl.ds(s * RPS + k * CH, CH)])
    plsc.subcore_barrier()

    ebase = (c * NSUB + s) * EPT2

    def chunk(k, _):
        base = ebase + k * CH
        pltpu.sync_copy(dst_hbm.at[pl.ds(base, CH)], idx_d)
        pltpu.sync_copy(ea_hbm.at[pl.ds(base, CH)], ea_buf)

        def crow(j, __):
            rows[j, pl.ds(0, DE)] = ea_buf[j, :]
            return __
        lax.fori_loop(0, CH, crow, None)
        pltpu.sync_copy(rows, accE.at[idx_d], add=True)
        return _

    lax.fori_loop(0, NFULL, chunk, None)

    tbase = ebase + NFULL * CH
    pltpu.sync_copy(dst_hbm.at[pl.ds(tbase, TAIL)], idx_t)
    pltpu.sync_copy(ea_hbm.at[pl.ds(tbase, TAIL)], ea_buf.at[pl.ds(0, TAIL)])

    def crow_t(j, _):
        rows[j, pl.ds(0, DE)] = ea_buf[j, :]
        return _
    lax.fori_loop(0, TAIL, crow_t, None)
    pltpu.sync_copy(rows.at[pl.ds(0, TAIL)], accE.at[idx_t], add=True)
    plsc.subcore_barrier()

    @pl.when(s < NSUB - 1)
    def _():
        pltpu.sync_copy(accE.at[pl.ds(s * RPS, RPS)],
                        eagg_out.at[pl.ds(c * N + s * RPS, RPS)])

    @pl.when(s == NSUB - 1)
    def _():
        pltpu.sync_copy(accE.at[pl.ds((NSUB - 1) * RPS, N - (NSUB - 1) * RPS)],
                        eagg_out.at[pl.ds(c * N + (NSUB - 1) * RPS,
                                          N - (NSUB - 1) * RPS)])


@functools.cache
def _get_s1b():
  return functools.partial(
    pl.kernel,
    out_type=jax.ShapeDtypeStruct((2 * N, DH), jnp.float32),
    mesh=plsc.VectorSubcoreMesh(core_axis_name="c", subcore_axis_name="s",
                                num_cores=NCORE, num_subcores=NSUB),
    scratch_types=[
        pltpu.VMEM((CH,), jnp.int32),
        pltpu.VMEM((TAIL,), jnp.int32),
        pltpu.VMEM((CH, DE), jnp.float32),
        pltpu.VMEM((CH, DH), jnp.float32),
        pltpu.VMEM_SHARED((NPAD, DH), jnp.float32),
        pltpu.SemaphoreType.DMA,
    ],
  )(_s1b_body)


def _s2_body(x_ref, a0_ref, a1_ref, e0_ref, e1_ref, we_ref, w1_ref, b1_ref,
             out_ref):
    agg = jnp.concatenate([a0_ref[0], a1_ref[0]], axis=-1)
    a = x_ref[...] + agg
    eagg = e0_ref[0] + e1_ref[0]
    we1 = jnp.dot(we_ref[...], w1_ref[...], preferred_element_type=jnp.float32)
    acc = jnp.dot(a, w1_ref[...], preferred_element_type=jnp.float32)
    acc = acc + jnp.dot(eagg, we1, preferred_element_type=jnp.float32)
    out_ref[...] = jnp.maximum(acc + b1_ref[...], 0.0)


def _s2(x, aggx2, eagg2, W_e, W1, b1):
    blk = 200
    grid = N // blk
    return pl.pallas_call(
        _s2_body,
        grid=(grid,),
        in_specs=[
            pl.BlockSpec((blk, D), lambda i: (i, 0)),
            pl.BlockSpec((1, blk, DH), lambda i: (0, i, 0)),
            pl.BlockSpec((1, blk, DH), lambda i: (1, i, 0)),
            pl.BlockSpec((1, blk, DE), lambda i: (0, i, 0)),
            pl.BlockSpec((1, blk, DE), lambda i: (1, i, 0)),
            pl.BlockSpec((DE, D), lambda i: (0, 0)),
            pl.BlockSpec((D, D), lambda i: (0, 0)),
            pl.BlockSpec((1, D), lambda i: (0, 0)),
        ],
        out_specs=pl.BlockSpec((blk, D), lambda i: (i, 0)),
        out_shape=jax.ShapeDtypeStruct((N, D), jnp.float32),
    )(x, aggx2, aggx2, eagg2, eagg2, W_e, W1, b1.reshape(1, D))


NOUT = NP + NC
RPT = NOUT // (NCORE * NSUB)
OCH = 64
CPT = NC // NSUB


def _s3_body(nrep_hbm, nidx_hbm, w_hbm, rseg_hbm, cidx_hbm, cseg_hbm,
             cw_hbm, out_hbm,
             cidxb, csegb, cwb, crows, tacc, iota16, cacc,
             nbuf, wbuf, rsegb, g, ob, accP, sem):
    c = lax.axis_index("c")
    s = lax.axis_index("s")
    zero16 = jnp.zeros((16,), jnp.float32)
    iota16[...] = lax.iota(jnp.int32, 16)

    def ztacc(i, _):
        for v in range(D // 16):
            tacc[i, pl.ds(v * 16, 16)] = zero16
        return _
    lax.fori_loop(0, B, ztacc, None)

    cb = s * CPT
    pltpu.sync_copy(cidx_hbm.at[pl.ds(cb, CPT)], cidxb)
    pltpu.sync_copy(cseg_hbm.at[pl.ds(cb, CPT)], csegb)
    pltpu.sync_copy(cw_hbm.at[pl.ds(cb, CPT)], cwb)
    pltpu.async_copy(nrep_hbm.at[cidxb], crows, sem).wait()

    def pool_row(j, _):
        segv = plsc.load_gather(csegb, [jnp.full((16,), j, jnp.int32)])
        wv = plsc.load_gather(cwb, [jnp.full((16,), j, jnp.int32)])
        for v in range(D // 16):
            val = crows[j, pl.ds(v * 16, 16)] * wv
            plsc.addupdate_scatter(tacc, [segv, iota16[...] + v * 16], val)
        return _
    lax.fori_loop(0, CPT, pool_row, None)

    pltpu.sync_copy(tacc, accP.at[s])

    def zcacc(i, _):
        for v in range(D // 16):
            cacc[i, pl.ds(v * 16, 16)] = zero16
        return _
    lax.fori_loop(0, 32, zcacc, None)
    plsc.subcore_barrier()
    for t in range(NSUB):
        pltpu.sync_copy(accP.at[t], tacc)

        def addrow(i, _):
            for v in range(D // 16):
                cacc[i, pl.ds(v * 16, 16)] = (cacc[i, pl.ds(v * 16, 16)]
                                              + tacc[i, pl.ds(v * 16, 16)])
            return _
        lax.fori_loop(0, B, addrow, None)

    rb = (c * NSUB + s) * RPT

    for ch in range(RPT // OCH):
        ob_base = rb + ch * OCH
        pltpu.sync_copy(nidx_hbm.at[pl.ds(ob_base, OCH)], nbuf)
        pltpu.sync_copy(w_hbm.at[pl.ds(ob_base, OCH)], wbuf)
        pltpu.sync_copy(rseg_hbm.at[pl.ds(ob_base, OCH)], rsegb)
        pltpu.async_copy(nrep_hbm.at[nbuf], g, sem).wait()

        def row(j, _):
            wv = plsc.load_gather(wbuf, [jnp.full((16,), j, jnp.int32)])
            rsv = plsc.load_gather(rsegb, [jnp.full((16,), j, jnp.int32)])
            for v in range(D // 16):
                ob[j, pl.ds(v * 16, 16)] = g[j, pl.ds(v * 16, 16)] * wv
            for v in range(D // 16):
                rv = plsc.load_gather(cacc, [rsv, iota16[...] + v * 16])
                ob[j, pl.ds(D + v * 16, 16)] = rv
            return _
        lax.fori_loop(0, OCH, row, None)
        pltpu.sync_copy(ob, out_hbm.at[pl.ds(ob_base, OCH)])


@functools.cache
def _get_s3():
  return functools.partial(
    pl.kernel,
    out_type=jax.ShapeDtypeStruct((NOUT, 2 * D), jnp.float32),
    mesh=plsc.VectorSubcoreMesh(core_axis_name="c", subcore_axis_name="s",
                                num_cores=NCORE, num_subcores=NSUB),
    compiler_params=pltpu.CompilerParams(needs_layout_passes=False),
    scratch_types=[
        pltpu.VMEM((CPT,), jnp.int32),
        pltpu.VMEM((CPT,), jnp.int32),
        pltpu.VMEM((CPT,), jnp.float32),
        pltpu.VMEM((CPT, D), jnp.float32),
        pltpu.VMEM((B, D), jnp.float32),
        pltpu.VMEM((16,), jnp.int32),
        pltpu.VMEM((32, D), jnp.float32),
        pltpu.VMEM((OCH,), jnp.int32),
        pltpu.VMEM((OCH,), jnp.float32),
        pltpu.VMEM((OCH,), jnp.int32),
        pltpu.VMEM((OCH, D), jnp.float32),
        pltpu.VMEM((OCH, 2 * D), jnp.float32),
        pltpu.VMEM_SHARED((NSUB, B, D), jnp.float32),
        pltpu.SemaphoreType.DMA,
    ],
  )(_s3_body)


def kernel(x, edge_index, edge_attr, pri_idx, pri_seg, cond_idx, cond_seg,
           W_e, W1, b1):
    src = edge_index[0].astype(jnp.int32)
    dst = edge_index[1].astype(jnp.int32)
    xr = x.reshape(N, 2, DH).transpose(1, 0, 2).reshape(2 * N, DH)
    sd = jnp.concatenate([src.reshape(E // CH, CH),
                          dst.reshape(E // CH, CH)], axis=1).reshape(-1)

    aggx = _get_s1()(sd, xr)
    eaggw = _get_s1b()(dst, edge_attr)
    eagg2 = eaggw.reshape(2, N, DH)[:, :, :DE]
    node_rep = _s2(x, aggx.reshape(2, N, DH), eagg2, W_e, W1, b1)

    pri_seg = pri_seg.astype(jnp.int32)
    cond_seg = cond_seg.astype(jnp.int32)
    bp = jnp.searchsorted(pri_seg, jnp.arange(B + 1, dtype=jnp.int32))
    bc = jnp.searchsorted(cond_seg, jnp.arange(B + 1, dtype=jnp.int32))
    cnt_p = (bp[1:] - bp[:-1]).astype(jnp.int32)
    cnt_c = (bc[1:] - bc[:-1]).astype(jnp.int32)
    start_p = bp[:-1].astype(jnp.int32)
    tot = cnt_p + cnt_c
    offsets = jnp.concatenate([jnp.zeros((1,), jnp.int32),
                               jnp.cumsum(tot)[:-1].astype(jnp.int32)])
    r = jnp.arange(NOUT, dtype=jnp.int32)
    seg_r = jnp.searchsorted(jnp.cumsum(tot).astype(jnp.int32), r,
                             side="right").astype(jnp.int32)
    local = r - offsets[seg_r]
    is_pri = local < cnt_p[seg_r]
    psrc = jnp.clip(start_p[seg_r] + local, 0, NP - 1)
    nidx = jnp.where(is_pri, pri_idx[psrc].astype(jnp.int32), 0)
    w = is_pri.astype(jnp.float32)
    rseg = jnp.where(is_pri, seg_r, 16).astype(jnp.int32)
    cw = (1.0 / jnp.maximum(cnt_c, 1).astype(jnp.float32))[cond_seg]

    out = _get_s3()(node_rep, nidx, w, rseg, cond_idx.astype(jnp.int32),
                    cond_seg, cw)
    return out


# speedup vs baseline: 2.2377x; 1.1099x over previous
import functools

import jax
import jax.numpy as jnp
from jax import lax
from jax.experimental import pallas as pl
from jax.experimental.pallas import tpu as pltpu
import jax.experimental.pallas.tpu_sc as plsc

N = 10000
E = 160000
D = 256
DE = 16
B = 16
NP = 6144
NC = 2048

NCORE = 2
NSUB = 16
DH = D // NCORE
EPT = E // NSUB
CH = 80
NCHUNK = EPT // CH
NPAD = 10240
RPS = NPAD // NSUB


def _s1_body(sd_hbm, xr_hbm, aggx_out, ir0, ir1, is0, is1, id0, id1,
             r0, r1, accA, semi, semg):
    idxraw = (ir0, ir1)
    idx_s = (is0, is1)
    idx_d = (id0, id1)
    rows = (r0, r1)
    c = lax.axis_index("c")
    s = lax.axis_index("s")
    zero16 = jnp.zeros((16,), jnp.float32)

    def zrow(i, _):
        for j in range(DH // 16):
            rows[0][i, pl.ds(j * 16, 16)] = zero16
        return _
    lax.fori_loop(0, CH, zrow, None)
    for k in range(RPS // CH):
        pltpu.sync_copy(rows[0], accA.at[pl.ds(s * RPS + k * CH, CH)])
    plsc.subcore_barrier()

    row0 = s * NCHUNK
    coff = c * N

    def load_idx(k, p):
        pltpu.async_copy(sd_hbm.at[pl.ds((row0 + k) * 2 * CH, 2 * CH)],
                         idxraw[p], semi)

    def build_idx(p):
        coffv = jnp.full((16,), coff, jnp.int32)
        for j in range(CH // 16):
            idx_s[p][pl.ds(j * 16, 16)] = (idxraw[p][pl.ds(j * 16, 16)]
                                           + coffv)
            idx_d[p][pl.ds(j * 16, 16)] = idxraw[p][pl.ds(CH + j * 16, 16)]

    load_idx(0, 0)
    pltpu.make_async_copy(sd_hbm.at[pl.ds(row0 * 2 * CH, 2 * CH)],
                          idxraw[0], semi).wait()
    build_idx(0)
    load_idx(1, 1)
    pltpu.async_copy(xr_hbm.at[idx_s[0]], rows[0], semg)

    def pair(kp, _):
        for par in (1, 0):
            k = 2 * kp + (1 if par == 1 else 2)
            q = 1 - par
            pltpu.make_async_copy(sd_hbm.at[pl.ds((row0 + k) * 2 * CH,
                                                  2 * CH)],
                                  idxraw[par], semi).wait()
            build_idx(par)
            load_idx(k + 1, q)
            pltpu.make_async_copy(xr_hbm.at[idx_s[q]], rows[q], semg).wait()
            pltpu.async_copy(xr_hbm.at[idx_s[par]], rows[par], semg)
            pltpu.sync_copy(rows[q], accA.at[idx_d[q]], add=True)
        return _

    lax.fori_loop(0, (NCHUNK - 1) // 2, pair, None)
    pltpu.make_async_copy(sd_hbm.at[pl.ds(row0 * 2 * CH, 2 * CH)],
                          idxraw[1], semi).wait()
    pltpu.make_async_copy(xr_hbm.at[idx_s[0]], rows[0], semg).wait()
    pltpu.sync_copy(rows[0], accA.at[idx_d[0]], add=True)
    plsc.subcore_barrier()

    @pl.when(s < NSUB - 1)
    def _():
        pltpu.sync_copy(accA.at[pl.ds(s * RPS, RPS)],
                        aggx_out.at[pl.ds(c * N + s * RPS, RPS)])

    @pl.when(s == NSUB - 1)
    def _():
        pltpu.sync_copy(accA.at[pl.ds((NSUB - 1) * RPS, N - (NSUB - 1) * RPS)],
                        aggx_out.at[pl.ds(c * N + (NSUB - 1) * RPS,
                                          N - (NSUB - 1) * RPS)])


@functools.cache
def _get_s1():
  return functools.partial(
    pl.kernel,
    out_type=jax.ShapeDtypeStruct((2 * N, DH), jnp.float32),
    mesh=plsc.VectorSubcoreMesh(core_axis_name="c", subcore_axis_name="s",
                                num_cores=NCORE, num_subcores=NSUB),
    scratch_types=[
        pltpu.VMEM((2 * CH,), jnp.int32),
        pltpu.VMEM((2 * CH,), jnp.int32),
        pltpu.VMEM((CH,), jnp.int32),
        pltpu.VMEM((CH,), jnp.int32),
        pltpu.VMEM((CH,), jnp.int32),
        pltpu.VMEM((CH,), jnp.int32),
        pltpu.VMEM((CH, DH), jnp.float32),
        pltpu.VMEM((CH, DH), jnp.float32),
        pltpu.VMEM_SHARED((NPAD, DH), jnp.float32),
        pltpu.SemaphoreType.DMA,
        pltpu.SemaphoreType.DMA,
    ],
  )(_s1_body)


EPT2 = E // (NCORE * NSUB)
NFULL = EPT2 // CH
TAIL = EPT2 - NFULL * CH


def _s1b_body(dst_hbm, ea_hbm, eagg_out, id0, id1, sx0, sx1, idx_t, ea0, ea1,
              r0, r1, accE, semi, semd):
    idx_d = (id0, id1)
    sidx = (sx0, sx1)
    ea_buf = (ea0, ea1)
    rows = (r0, r1)
    c = lax.axis_index("c")
    s = lax.axis_index("s")
    zero16 = jnp.zeros((16,), jnp.float32)

    for p in (0, 1):
        def zrow(i, _, p=p):
            for j in range(DH // 16):
                rows[p][i, pl.ds(j * 16, 16)] = zero16
            return _
        lax.fori_loop(0, CH, zrow, None)
    for k in range(RPS // CH):
        pltpu.sync_copy(rows[0], accE.at[pl.ds(s * RPS + k * CH, CH)])
    plsc.subcore_barrier()

    ebase = (c * NSUB + s) * EPT2

    def load(k, p):
        pltpu.async_copy(dst_hbm.at[pl.ds(ebase + k * CH, CH)], idx_d[p],
                         semi)
        pltpu.async_copy(ea_hbm.at[pl.ds(ebase + k * CH, CH)], ea_buf[p],
                         semi)

    def wait_load(p):
        pltpu.make_async_copy(dst_hbm.at[pl.ds(ebase, CH)], idx_d[p],
                              semi).wait()
        pltpu.make_async_copy(ea_hbm.at[pl.ds(ebase, CH)], ea_buf[p],
                              semi).wait()

    def crow(p):
        def body(j, _):
            rows[p][j, pl.ds(0, DE)] = ea_buf[p][j, :]
            return _
        lax.fori_loop(0, CH, body, None)
        for j in range(CH // 16):
            sidx[p][pl.ds(j * 16, 16)] = idx_d[p][pl.ds(j * 16, 16)]

    def wait_scat(p):
        pltpu.make_async_copy(rows[p], accE.at[sidx[p]], semd).wait()

    load(0, 0)
    wait_load(0)
    load(1, 1)
    crow(0)
    pltpu.async_copy(rows[0], accE.at[sidx[0]], semd, add=True)

    def pair(kp, _):
        for par in (1, 0):
            k = 2 * kp + (1 if par == 1 else 2)
            q = 1 - par
            wait_load(par)

            @pl.when(k + 1 < NFULL)
            def _():
                load(k + 1, q)
            crow(par)
            wait_scat(q)
            pltpu.async_copy(rows[par], accE.at[sidx[par]], semd, add=True)
        return _

    lax.fori_loop(0, (NFULL - 1) // 2, pair, None)
    wait_load(1)
    crow(1)
    wait_scat(0)
    pltpu.async_copy(rows[1], accE.at[sidx[1]], semd, add=True)

    tbase = ebase + NFULL * CH
    pltpu.sync_copy(dst_hbm.at[pl.ds(tbase, TAIL)], idx_t)
    pltpu.sync_copy(ea_hbm.at[pl.ds(tbase, TAIL)], ea0.at[pl.ds(0, TAIL)])
    wait_scat(1)

    def crow_t(j, _):
        r0[j, pl.ds(0, DE)] = ea0[j, :]
        return _
    lax.fori_loop(0, TAIL, crow_t, None)
    pltpu.sync_copy(r0.at[pl.ds(0, TAIL)], accE.at[idx_t], add=True)
    plsc.subcore_barrier()

    @pl.when(s < NSUB - 1)
    def _():
        pltpu.sync_copy(accE.at[pl.ds(s * RPS, RPS)],
                        eagg_out.at[pl.ds(c * N + s * RPS, RPS)])

    @pl.when(s == NSUB - 1)
    def _():
        pltpu.sync_copy(accE.at[pl.ds((NSUB - 1) * RPS, N - (NSUB - 1) * RPS)],
                        eagg_out.at[pl.ds(c * N + (NSUB - 1) * RPS,
                                          N - (NSUB - 1) * RPS)])


@functools.cache
def _get_s1b():
  return functools.partial(
    pl.kernel,
    out_type=jax.ShapeDtypeStruct((2 * N, DH), jnp.float32),
    mesh=plsc.VectorSubcoreMesh(core_axis_name="c", subcore_axis_name="s",
                                num_cores=NCORE, num_subcores=NSUB),
    scratch_types=[
        pltpu.VMEM((CH,), jnp.int32),
        pltpu.VMEM((CH,), jnp.int32),
        pltpu.VMEM((CH,), jnp.int32),
        pltpu.VMEM((CH,), jnp.int32),
        pltpu.VMEM((TAIL,), jnp.int32),
        pltpu.VMEM((CH, DE), jnp.float32),
        pltpu.VMEM((CH, DE), jnp.float32),
        pltpu.VMEM((CH, DH), jnp.float32),
        pltpu.VMEM((CH, DH), jnp.float32),
        pltpu.VMEM_SHARED((NPAD, DH), jnp.float32),
        pltpu.SemaphoreType.DMA,
        pltpu.SemaphoreType.DMA,
    ],
  )(_s1b_body)


def _s2_body(x_ref, a0_ref, a1_ref, e0_ref, e1_ref, we_ref, w1_ref, b1_ref,
             out_ref):
    agg = jnp.concatenate([a0_ref[0], a1_ref[0]], axis=-1)
    a = x_ref[...] + agg
    eagg = e0_ref[0] + e1_ref[0]
    we1 = jnp.dot(we_ref[...], w1_ref[...], preferred_element_type=jnp.float32)
    acc = jnp.dot(a, w1_ref[...], preferred_element_type=jnp.float32)
    acc = acc + jnp.dot(eagg, we1, preferred_element_type=jnp.float32)
    out_ref[...] = jnp.maximum(acc + b1_ref[...], 0.0)


def _s2(x, aggx2, eagg2, W_e, W1, b1):
    blk = 200
    grid = N // blk
    return pl.pallas_call(
        _s2_body,
        grid=(grid,),
        in_specs=[
            pl.BlockSpec((blk, D), lambda i: (i, 0)),
            pl.BlockSpec((1, blk, DH), lambda i: (0, i, 0)),
            pl.BlockSpec((1, blk, DH), lambda i: (1, i, 0)),
            pl.BlockSpec((1, blk, DE), lambda i: (0, i, 0)),
            pl.BlockSpec((1, blk, DE), lambda i: (1, i, 0)),
            pl.BlockSpec((DE, D), lambda i: (0, 0)),
            pl.BlockSpec((D, D), lambda i: (0, 0)),
            pl.BlockSpec((1, D), lambda i: (0, 0)),
        ],
        out_specs=pl.BlockSpec((blk, D), lambda i: (i, 0)),
        out_shape=jax.ShapeDtypeStruct((N, D), jnp.float32),
    )(x, aggx2, aggx2, eagg2, eagg2, W_e, W1, b1.reshape(1, D))


NOUT = NP + NC
RPT = NOUT // (NCORE * NSUB)
OCH = 64
CPT = NC // NSUB


def _s3_body(nrep_hbm, nidx_hbm, w_hbm, rseg_hbm, cidx_hbm, cseg_hbm,
             cw_hbm, out_hbm,
             cidxb, csegb, cwb, crows, tacc, iota16, cacc,
             nbuf, wbuf, rsegb, g, ob, accP, sem):
    c = lax.axis_index("c")
    s = lax.axis_index("s")
    zero16 = jnp.zeros((16,), jnp.float32)
    iota16[...] = lax.iota(jnp.int32, 16)

    def ztacc(i, _):
        for v in range(D // 16):
            tacc[i, pl.ds(v * 16, 16)] = zero16
        return _
    lax.fori_loop(0, B, ztacc, None)

    cb = s * CPT
    pltpu.sync_copy(cidx_hbm.at[pl.ds(cb, CPT)], cidxb)
    pltpu.sync_copy(cseg_hbm.at[pl.ds(cb, CPT)], csegb)
    pltpu.sync_copy(cw_hbm.at[pl.ds(cb, CPT)], cwb)
    pltpu.async_copy(nrep_hbm.at[cidxb], crows, sem).wait()

    def pool_row(j, _):
        segv = plsc.load_gather(csegb, [jnp.full((16,), j, jnp.int32)])
        wv = plsc.load_gather(cwb, [jnp.full((16,), j, jnp.int32)])
        for v in range(D // 16):
            val = crows[j, pl.ds(v * 16, 16)] * wv
            plsc.addupdate_scatter(tacc, [segv, iota16[...] + v * 16], val)
        return _
    lax.fori_loop(0, CPT, pool_row, None)

    pltpu.sync_copy(tacc, accP.at[s])

    def zcacc(i, _):
        for v in range(D // 16):
            cacc[i, pl.ds(v * 16, 16)] = zero16
        return _
    lax.fori_loop(0, 32, zcacc, None)
    plsc.subcore_barrier()
    for t in range(NSUB):
        pltpu.sync_copy(accP.at[t], tacc)

        def addrow(i, _):
            for v in range(D // 16):
                cacc[i, pl.ds(v * 16, 16)] = (cacc[i, pl.ds(v * 16, 16)]
                                              + tacc[i, pl.ds(v * 16, 16)])
            return _
        lax.fori_loop(0, B, addrow, None)

    rb = (c * NSUB + s) * RPT

    for ch in range(RPT // OCH):
        ob_base = rb + ch * OCH
        pltpu.sync_copy(nidx_hbm.at[pl.ds(ob_base, OCH)], nbuf)
        pltpu.sync_copy(w_hbm.at[pl.ds(ob_base, OCH)], wbuf)
        pltpu.sync_copy(rseg_hbm.at[pl.ds(ob_base, OCH)], rsegb)
        pltpu.async_copy(nrep_hbm.at[nbuf], g, sem).wait()

        def row(j, _):
            wv = plsc.load_gather(wbuf, [jnp.full((16,), j, jnp.int32)])
            rsv = plsc.load_gather(rsegb, [jnp.full((16,), j, jnp.int32)])
            for v in range(D // 16):
                ob[j, pl.ds(v * 16, 16)] = g[j, pl.ds(v * 16, 16)] * wv
            for v in range(D // 16):
                rv = plsc.load_gather(cacc, [rsv, iota16[...] + v * 16])
                ob[j, pl.ds(D + v * 16, 16)] = rv
            return _
        lax.fori_loop(0, OCH, row, None)
        pltpu.sync_copy(ob, out_hbm.at[pl.ds(ob_base, OCH)])


@functools.cache
def _get_s3():
  return functools.partial(
    pl.kernel,
    out_type=jax.ShapeDtypeStruct((NOUT, 2 * D), jnp.float32),
    mesh=plsc.VectorSubcoreMesh(core_axis_name="c", subcore_axis_name="s",
                                num_cores=NCORE, num_subcores=NSUB),
    compiler_params=pltpu.CompilerParams(needs_layout_passes=False),
    scratch_types=[
        pltpu.VMEM((CPT,), jnp.int32),
        pltpu.VMEM((CPT,), jnp.int32),
        pltpu.VMEM((CPT,), jnp.float32),
        pltpu.VMEM((CPT, D), jnp.float32),
        pltpu.VMEM((B, D), jnp.float32),
        pltpu.VMEM((16,), jnp.int32),
        pltpu.VMEM((32, D), jnp.float32),
        pltpu.VMEM((OCH,), jnp.int32),
        pltpu.VMEM((OCH,), jnp.float32),
        pltpu.VMEM((OCH,), jnp.int32),
        pltpu.VMEM((OCH, D), jnp.float32),
        pltpu.VMEM((OCH, 2 * D), jnp.float32),
        pltpu.VMEM_SHARED((NSUB, B, D), jnp.float32),
        pltpu.SemaphoreType.DMA,
    ],
  )(_s3_body)


def kernel(x, edge_index, edge_attr, pri_idx, pri_seg, cond_idx, cond_seg,
           W_e, W1, b1):
    src = edge_index[0].astype(jnp.int32)
    dst = edge_index[1].astype(jnp.int32)
    xr = x.reshape(N, 2, DH).transpose(1, 0, 2).reshape(2 * N, DH)
    sd = jnp.concatenate([src.reshape(E // CH, CH),
                          dst.reshape(E // CH, CH)], axis=1).reshape(-1)

    aggx = _get_s1()(sd, xr)
    eaggw = _get_s1b()(dst, edge_attr)
    eagg2 = eaggw.reshape(2, N, DH)[:, :, :DE]
    node_rep = _s2(x, aggx.reshape(2, N, DH), eagg2, W_e, W1, b1)

    pri_seg = pri_seg.astype(jnp.int32)
    cond_seg = cond_seg.astype(jnp.int32)
    bp = jnp.searchsorted(pri_seg, jnp.arange(B + 1, dtype=jnp.int32))
    bc = jnp.searchsorted(cond_seg, jnp.arange(B + 1, dtype=jnp.int32))
    cnt_p = (bp[1:] - bp[:-1]).astype(jnp.int32)
    cnt_c = (bc[1:] - bc[:-1]).astype(jnp.int32)
    start_p = bp[:-1].astype(jnp.int32)
    tot = cnt_p + cnt_c
    offsets = jnp.concatenate([jnp.zeros((1,), jnp.int32),
                               jnp.cumsum(tot)[:-1].astype(jnp.int32)])
    r = jnp.arange(NOUT, dtype=jnp.int32)
    seg_r = jnp.searchsorted(jnp.cumsum(tot).astype(jnp.int32), r,
                             side="right").astype(jnp.int32)
    local = r - offsets[seg_r]
    is_pri = local < cnt_p[seg_r]
    psrc = jnp.clip(start_p[seg_r] + local, 0, NP - 1)
    nidx = jnp.where(is_pri, pri_idx[psrc].astype(jnp.int32), 0)
    w = is_pri.astype(jnp.float32)
    rseg = jnp.where(is_pri, seg_r, 16).astype(jnp.int32)
    cw = (1.0 / jnp.maximum(cnt_c, 1).astype(jnp.float32))[cond_seg]

    out = _get_s3()(node_rep, nidx, w, rseg, cond_idx.astype(jnp.int32),
                    cond_seg, cw)
    return out


# speedup vs baseline: 2.3493x; 1.0499x over previous
import functools

import jax
import jax.numpy as jnp
from jax import lax
from jax.experimental import pallas as pl
from jax.experimental.pallas import tpu as pltpu
import jax.experimental.pallas.tpu_sc as plsc

N = 10000
E = 160000
D = 256
DE = 16
B = 16
NP = 6144
NC = 2048

NCORE = 2
NSUB = 16
DH = D // NCORE
EPT = E // NSUB
CH = 80
NCHUNK = EPT // CH
NPAD = 10240
RPS = NPAD // NSUB


def _s1_body(sd_hbm, xr_hbm, aggx_out, ir0, ir1, is0, is1, id0, id1,
             r0, r1, accA, semi, semg):
    idxraw = (ir0, ir1)
    idx_s = (is0, is1)
    idx_d = (id0, id1)
    rows = (r0, r1)
    c = lax.axis_index("c")
    s = lax.axis_index("s")
    zero16 = jnp.zeros((16,), jnp.float32)

    def zrow(i, _):
        for j in range(DH // 16):
            rows[0][i, pl.ds(j * 16, 16)] = zero16
        return _
    lax.fori_loop(0, CH, zrow, None)
    for k in range(RPS // CH):
        pltpu.sync_copy(rows[0], accA.at[pl.ds(s * RPS + k * CH, CH)])
    plsc.subcore_barrier()

    row0 = s * NCHUNK
    coff = c * N

    def load_idx(k, p):
        pltpu.async_copy(sd_hbm.at[pl.ds((row0 + k) * 2 * CH, 2 * CH)],
                         idxraw[p], semi)

    def build_idx(p):
        coffv = jnp.full((16,), coff, jnp.int32)
        for j in range(CH // 16):
            idx_s[p][pl.ds(j * 16, 16)] = (idxraw[p][pl.ds(j * 16, 16)]
                                           + coffv)
            idx_d[p][pl.ds(j * 16, 16)] = idxraw[p][pl.ds(CH + j * 16, 16)]

    load_idx(0, 0)
    pltpu.make_async_copy(sd_hbm.at[pl.ds(row0 * 2 * CH, 2 * CH)],
                          idxraw[0], semi).wait()
    build_idx(0)
    load_idx(1, 1)
    pltpu.async_copy(xr_hbm.at[idx_s[0]], rows[0], semg)

    def pair(kp, _):
        for par in (1, 0):
            k = 2 * kp + (1 if par == 1 else 2)
            q = 1 - par
            pltpu.make_async_copy(sd_hbm.at[pl.ds((row0 + k) * 2 * CH,
                                                  2 * CH)],
                                  idxraw[par], semi).wait()
            build_idx(par)
            load_idx(k + 1, q)
            pltpu.make_async_copy(xr_hbm.at[idx_s[q]], rows[q], semg).wait()
            pltpu.async_copy(xr_hbm.at[idx_s[par]], rows[par], semg)
            pltpu.sync_copy(rows[q], accA.at[idx_d[q]], add=True)
        return _

    lax.fori_loop(0, (NCHUNK - 1) // 2, pair, None)
    pltpu.make_async_copy(sd_hbm.at[pl.ds(row0 * 2 * CH, 2 * CH)],
                          idxraw[1], semi).wait()
    pltpu.make_async_copy(xr_hbm.at[idx_s[0]], rows[0], semg).wait()
    pltpu.sync_copy(rows[0], accA.at[idx_d[0]], add=True)
    plsc.subcore_barrier()

    @pl.when(s < NSUB - 1)
    def _():
        pltpu.sync_copy(accA.at[pl.ds(s * RPS, RPS)],
                        aggx_out.at[pl.ds(c * N + s * RPS, RPS)])

    @pl.when(s == NSUB - 1)
    def _():
        pltpu.sync_copy(accA.at[pl.ds((NSUB - 1) * RPS, N - (NSUB - 1) * RPS)],
                        aggx_out.at[pl.ds(c * N + (NSUB - 1) * RPS,
                                          N - (NSUB - 1) * RPS)])


@functools.cache
def _get_s1():
  return functools.partial(
    pl.kernel,
    out_type=jax.ShapeDtypeStruct((2 * N, DH), jnp.float32),
    mesh=plsc.VectorSubcoreMesh(core_axis_name="c", subcore_axis_name="s",
                                num_cores=NCORE, num_subcores=NSUB),
    scratch_types=[
        pltpu.VMEM((2 * CH,), jnp.int32),
        pltpu.VMEM((2 * CH,), jnp.int32),
        pltpu.VMEM((CH,), jnp.int32),
        pltpu.VMEM((CH,), jnp.int32),
        pltpu.VMEM((CH,), jnp.int32),
        pltpu.VMEM((CH,), jnp.int32),
        pltpu.VMEM((CH, DH), jnp.float32),
        pltpu.VMEM((CH, DH), jnp.float32),
        pltpu.VMEM_SHARED((NPAD, DH), jnp.float32),
        pltpu.SemaphoreType.DMA,
        pltpu.SemaphoreType.DMA,
    ],
  )(_s1_body)


EPT2 = E // (NCORE * NSUB)
NFULL = EPT2 // CH
TAIL = EPT2 - NFULL * CH


def _s1b_body(dst_hbm, ea_hbm, eagg_out, id0, id1, sx0, sx1, idx_t, ea0, ea1,
              r0, r1, accE, semi, semd):
    idx_d = (id0, id1)
    sidx = (sx0, sx1)
    ea_buf = (ea0, ea1)
    rows = (r0, r1)
    c = lax.axis_index("c")
    s = lax.axis_index("s")
    zero16 = jnp.zeros((16,), jnp.float32)

    for p in (0, 1):
        def zrow(i, _, p=p):
            for j in range(DH // 16):
                rows[p][i, pl.ds(j * 16, 16)] = zero16
            return _
        lax.fori_loop(0, CH, zrow, None)
    for k in range(RPS // CH):
        pltpu.sync_copy(rows[0], accE.at[pl.ds(s * RPS + k * CH, CH)])
    plsc.subcore_barrier()

    ebase = (c * NSUB + s) * EPT2

    def load(k, p):
        pltpu.async_copy(dst_hbm.at[pl.ds(ebase + k * CH, CH)], idx_d[p],
                         semi)
        pltpu.async_copy(ea_hbm.at[pl.ds(ebase + k * CH, CH)], ea_buf[p],
                         semi)

    def wait_load(p):
        pltpu.make_async_copy(dst_hbm.at[pl.ds(ebase, CH)], idx_d[p],
                              semi).wait()
        pltpu.make_async_copy(ea_hbm.at[pl.ds(ebase, CH)], ea_buf[p],
                              semi).wait()

    def crow(p):
        def body(j, _):
            rows[p][j, pl.ds(0, DE)] = ea_buf[p][j, :]
            return _
        lax.fori_loop(0, CH, body, None)
        for j in range(CH // 16):
            sidx[p][pl.ds(j * 16, 16)] = idx_d[p][pl.ds(j * 16, 16)]

    def wait_scat(p):
        pltpu.make_async_copy(rows[p], accE.at[sidx[p]], semd).wait()

    load(0, 0)
    wait_load(0)
    load(1, 1)
    crow(0)
    pltpu.async_copy(rows[0], accE.at[sidx[0]], semd, add=True)

    def pair(kp, _):
        for par in (1, 0):
            k = 2 * kp + (1 if par == 1 else 2)
            q = 1 - par
            wait_load(par)

            @pl.when(k + 1 < NFULL)
            def _():
                load(k + 1, q)
            crow(par)
            wait_scat(q)
            pltpu.async_copy(rows[par], accE.at[sidx[par]], semd, add=True)
        return _

    lax.fori_loop(0, (NFULL - 1) // 2, pair, None)
    wait_load(1)
    crow(1)
    wait_scat(0)
    pltpu.async_copy(rows[1], accE.at[sidx[1]], semd, add=True)

    tbase = ebase + NFULL * CH
    pltpu.sync_copy(dst_hbm.at[pl.ds(tbase, TAIL)], idx_t)
    pltpu.sync_copy(ea_hbm.at[pl.ds(tbase, TAIL)], ea0.at[pl.ds(0, TAIL)])
    wait_scat(1)

    def crow_t(j, _):
        r0[j, pl.ds(0, DE)] = ea0[j, :]
        return _
    lax.fori_loop(0, TAIL, crow_t, None)
    pltpu.sync_copy(r0.at[pl.ds(0, TAIL)], accE.at[idx_t], add=True)
    plsc.subcore_barrier()

    @pl.when(s < NSUB - 1)
    def _():
        pltpu.sync_copy(accE.at[pl.ds(s * RPS, RPS)],
                        eagg_out.at[pl.ds(c * N + s * RPS, RPS)])

    @pl.when(s == NSUB - 1)
    def _():
        pltpu.sync_copy(accE.at[pl.ds((NSUB - 1) * RPS, N - (NSUB - 1) * RPS)],
                        eagg_out.at[pl.ds(c * N + (NSUB - 1) * RPS,
                                          N - (NSUB - 1) * RPS)])


@functools.cache
def _get_s1b():
  return functools.partial(
    pl.kernel,
    out_type=jax.ShapeDtypeStruct((2 * N, DH), jnp.float32),
    mesh=plsc.VectorSubcoreMesh(core_axis_name="c", subcore_axis_name="s",
                                num_cores=NCORE, num_subcores=NSUB),
    scratch_types=[
        pltpu.VMEM((CH,), jnp.int32),
        pltpu.VMEM((CH,), jnp.int32),
        pltpu.VMEM((CH,), jnp.int32),
        pltpu.VMEM((CH,), jnp.int32),
        pltpu.VMEM((TAIL,), jnp.int32),
        pltpu.VMEM((CH, DE), jnp.float32),
        pltpu.VMEM((CH, DE), jnp.float32),
        pltpu.VMEM((CH, DH), jnp.float32),
        pltpu.VMEM((CH, DH), jnp.float32),
        pltpu.VMEM_SHARED((NPAD, DH), jnp.float32),
        pltpu.SemaphoreType.DMA,
        pltpu.SemaphoreType.DMA,
    ],
  )(_s1b_body)


def _s2_body(x_ref, a0_ref, a1_ref, e0_ref, e1_ref, we_ref, w1_ref, b1_ref,
             out_ref):
    agg = jnp.concatenate([a0_ref[0], a1_ref[0]], axis=-1)
    a = x_ref[...] + agg
    eagg = e0_ref[0] + e1_ref[0]
    we1 = jnp.dot(we_ref[...], w1_ref[...], preferred_element_type=jnp.float32)
    acc = jnp.dot(a, w1_ref[...], preferred_element_type=jnp.float32)
    acc = acc + jnp.dot(eagg, we1, preferred_element_type=jnp.float32)
    out_ref[...] = jnp.maximum(acc + b1_ref[...], 0.0)


def _s2(x, aggx2, eagg2, W_e, W1, b1):
    blk = 200
    grid = N // blk
    return pl.pallas_call(
        _s2_body,
        grid=(grid,),
        in_specs=[
            pl.BlockSpec((blk, D), lambda i: (i, 0)),
            pl.BlockSpec((1, blk, DH), lambda i: (0, i, 0)),
            pl.BlockSpec((1, blk, DH), lambda i: (1, i, 0)),
            pl.BlockSpec((1, blk, DE), lambda i: (0, i, 0)),
            pl.BlockSpec((1, blk, DE), lambda i: (1, i, 0)),
            pl.BlockSpec((DE, D), lambda i: (0, 0)),
            pl.BlockSpec((D, D), lambda i: (0, 0)),
            pl.BlockSpec((1, D), lambda i: (0, 0)),
        ],
        out_specs=pl.BlockSpec((blk, D), lambda i: (i, 0)),
        out_shape=jax.ShapeDtypeStruct((N, D), jnp.float32),
    )(x, aggx2, aggx2, eagg2, eagg2, W_e, W1, b1.reshape(1, D))


NOUT = NP + NC
RPT = NOUT // (NCORE * NSUB)
OCH = 32
CPT = NC // NSUB


def _s3_body(nrep_hbm, comba_hbm, combb_hbm, out_hbm,
             comba, crows, tacc, iota16, cacc,
             cb0, cb1, g0, g1, ob0, ob1, accP, semg, semo):
    cbufs = (cb0, cb1)
    g = (g0, g1)
    ob = (ob0, ob1)
    c = lax.axis_index("c")
    s = lax.axis_index("s")
    zero16 = jnp.zeros((16,), jnp.float32)
    iota16[...] = lax.iota(jnp.int32, 16)

    def ztacc(i, _):
        for v in range(D // 16):
            tacc[i, pl.ds(v * 16, 16)] = zero16
        return _
    lax.fori_loop(0, B, ztacc, None)

    pltpu.sync_copy(comba_hbm.at[pl.ds(s * 3 * CPT, 3 * CPT)], comba)
    pltpu.async_copy(nrep_hbm.at[comba.at[pl.ds(0, CPT)]], crows,
                     semg).wait()

    def pool_row(j, _):
        segv = plsc.load_gather(comba, [jnp.full((16,), CPT, jnp.int32) + j])
        wv = plsc.bitcast(
            plsc.load_gather(comba,
                             [jnp.full((16,), 2 * CPT, jnp.int32) + j]),
            jnp.float32)
        for v in range(D // 16):
            val = crows[j, pl.ds(v * 16, 16)] * wv
            plsc.addupdate_scatter(tacc, [segv, iota16[...] + v * 16], val)
        return _
    lax.fori_loop(0, CPT, pool_row, None)

    pltpu.sync_copy(tacc, accP.at[s])

    def zcacc(i, _):
        for v in range(D // 16):
            cacc[i, pl.ds(v * 16, 16)] = zero16
        return _
    lax.fori_loop(0, 32, zcacc, None)
    plsc.subcore_barrier()
    for t in range(NSUB):
        pltpu.sync_copy(accP.at[t], tacc)

        def addrow(i, _):
            for v in range(D // 16):
                cacc[i, pl.ds(v * 16, 16)] = (cacc[i, pl.ds(v * 16, 16)]
                                              + tacc[i, pl.ds(v * 16, 16)])
            return _
        lax.fori_loop(0, B, addrow, None)

    NCH = RPT // OCH
    rb = (c * NSUB + s) * RPT
    rbch = (c * NSUB + s) * NCH

    def compose(p):
        def row(j, _):
            rsv = plsc.load_gather(cbufs[p],
                                   [jnp.full((16,), OCH, jnp.int32) + j])
            wv = plsc.bitcast(
                plsc.load_gather(cbufs[p],
                                 [jnp.full((16,), 2 * OCH, jnp.int32) + j]),
                jnp.float32)
            for v in range(D // 16):
                ob[p][j, pl.ds(v * 16, 16)] = g[p][j, pl.ds(v * 16, 16)] * wv
            for v in range(D // 16):
                rv = plsc.load_gather(cacc, [rsv, iota16[...] + v * 16])
                ob[p][j, pl.ds(D + v * 16, 16)] = rv
            return _
        lax.fori_loop(0, OCH, row, None)

    pltpu.sync_copy(combb_hbm.at[pl.ds(rbch * 3 * OCH, 3 * OCH)], cb0)
    pltpu.async_copy(nrep_hbm.at[cb0.at[pl.ds(0, OCH)]], g0, semg)
    for ch in range(NCH):
        p = ch % 2
        q = 1 - p
        if ch + 1 < NCH:
            pltpu.sync_copy(combb_hbm.at[pl.ds((rbch + ch + 1) * 3 * OCH,
                                               3 * OCH)], cbufs[q])
        pltpu.make_async_copy(nrep_hbm.at[cbufs[p].at[pl.ds(0, OCH)]],
                              g[p], semg).wait()
        if ch + 1 < NCH:
            pltpu.async_copy(nrep_hbm.at[cbufs[q].at[pl.ds(0, OCH)]],
                             g[q], semg)
        if ch >= 2:
            pltpu.make_async_copy(ob[p],
                                  out_hbm.at[pl.ds(rb + (ch - 2) * OCH, OCH)],
                                  semo).wait()
        compose(p)
        pltpu.async_copy(ob[p], out_hbm.at[pl.ds(rb + ch * OCH, OCH)], semo)
    pltpu.make_async_copy(ob0, out_hbm.at[pl.ds(rb + (NCH - 2) * OCH, OCH)],
                          semo).wait()
    pltpu.make_async_copy(ob1, out_hbm.at[pl.ds(rb + (NCH - 1) * OCH, OCH)],
                          semo).wait()


@functools.cache
def _get_s3():
  return functools.partial(
    pl.kernel,
    out_type=jax.ShapeDtypeStruct((NOUT, 2 * D), jnp.float32),
    mesh=plsc.VectorSubcoreMesh(core_axis_name="c", subcore_axis_name="s",
                                num_cores=NCORE, num_subcores=NSUB),
    compiler_params=pltpu.CompilerParams(needs_layout_passes=False),
    scratch_types=[
        pltpu.VMEM((3 * CPT,), jnp.int32),
        pltpu.VMEM((CPT, D), jnp.float32),
        pltpu.VMEM((B, D), jnp.float32),
        pltpu.VMEM((16,), jnp.int32),
        pltpu.VMEM((32, D), jnp.float32),
        pltpu.VMEM((3 * OCH,), jnp.int32),
        pltpu.VMEM((3 * OCH,), jnp.int32),
        pltpu.VMEM((OCH, D), jnp.float32),
        pltpu.VMEM((OCH, D), jnp.float32),
        pltpu.VMEM((OCH, 2 * D), jnp.float32),
        pltpu.VMEM((OCH, 2 * D), jnp.float32),
        pltpu.VMEM_SHARED((NSUB, B, D), jnp.float32),
        pltpu.SemaphoreType.DMA,
        pltpu.SemaphoreType.DMA,
    ],
  )(_s3_body)


def kernel(x, edge_index, edge_attr, pri_idx, pri_seg, cond_idx, cond_seg,
           W_e, W1, b1):
    src = edge_index[0].astype(jnp.int32)
    dst = edge_index[1].astype(jnp.int32)
    xr = x.reshape(N, 2, DH).transpose(1, 0, 2).reshape(2 * N, DH)
    sd = jnp.concatenate([src.reshape(E // CH, CH),
                          dst.reshape(E // CH, CH)], axis=1).reshape(-1)

    aggx = _get_s1()(sd, xr)
    eaggw = _get_s1b()(dst, edge_attr)
    eagg2 = eaggw.reshape(2, N, DH)[:, :, :DE]
    node_rep = _s2(x, aggx.reshape(2, N, DH), eagg2, W_e, W1, b1)

    pri_seg = pri_seg.astype(jnp.int32)
    cond_seg = cond_seg.astype(jnp.int32)
    bp = jnp.searchsorted(pri_seg, jnp.arange(B + 1, dtype=jnp.int32))
    bc = jnp.searchsorted(cond_seg, jnp.arange(B + 1, dtype=jnp.int32))
    cnt_p = (bp[1:] - bp[:-1]).astype(jnp.int32)
    cnt_c = (bc[1:] - bc[:-1]).astype(jnp.int32)
    start_p = bp[:-1].astype(jnp.int32)
    tot = cnt_p + cnt_c
    offsets = jnp.concatenate([jnp.zeros((1,), jnp.int32),
                               jnp.cumsum(tot)[:-1].astype(jnp.int32)])
    r = jnp.arange(NOUT, dtype=jnp.int32)
    seg_r = jnp.searchsorted(jnp.cumsum(tot).astype(jnp.int32), r,
                             side="right").astype(jnp.int32)
    local = r - offsets[seg_r]
    is_pri = local < cnt_p[seg_r]
    psrc = jnp.clip(start_p[seg_r] + local, 0, NP - 1)
    nidx = jnp.where(is_pri, pri_idx[psrc].astype(jnp.int32), 0)
    w = is_pri.astype(jnp.float32)
    rseg = jnp.where(is_pri, seg_r, 16).astype(jnp.int32)
    cw = (1.0 / jnp.maximum(cnt_c, 1).astype(jnp.float32))[cond_seg]

    comba = jnp.concatenate(
        [cond_idx.astype(jnp.int32).reshape(NSUB, CPT),
         cond_seg.reshape(NSUB, CPT),
         jax.lax.bitcast_convert_type(cw, jnp.int32).reshape(NSUB, CPT)],
        axis=1).reshape(-1)
    combb = jnp.concatenate(
        [nidx.reshape(NOUT // OCH, OCH),
         rseg.reshape(NOUT // OCH, OCH),
         jax.lax.bitcast_convert_type(w, jnp.int32).reshape(NOUT // OCH,
                                                            OCH)],
        axis=1).reshape(-1)

    out = _get_s3()(node_rep, comba, combb)
    return out
